# baseline (speedup 1.0000x reference)
import functools

import jax
import jax.numpy as jnp
from jax import lax
from jax.experimental import pallas as pl
from jax.experimental.pallas import tpu as pltpu
from jax.experimental.pallas import tpu_sc as plsc

B0, H, D = 4096, 50, 64
B = B0 * H
NC, NS = 2, 16
NW = NC * NS
BPW = B // NW
CHUNK = 800
NCHUNK = BPW // CHUNK

_MESH = plsc.VectorSubcoreMesh(core_axis_name="c", subcore_axis_name="s")


@functools.partial(
    pl.kernel,
    out_type=jax.ShapeDtypeStruct((B, D), jnp.float32),
    mesh=_MESH,
    scratch_types=[
        pltpu.VMEM((BPW,), jnp.int32),
        pltpu.VMEM((CHUNK, D), jnp.float32),
        pltpu.VMEM((CHUNK, D), jnp.float32),
        pltpu.SemaphoreType.DMA,
        pltpu.SemaphoreType.DMA,
        pltpu.SemaphoreType.DMA,
        pltpu.SemaphoreType.DMA,
    ],
    compiler_params=pltpu.CompilerParams(use_tc_tiling_on_sc=False),
)
def _gather_rows(idx_hbm, table_hbm, out_hbm, idx_v, r0, r1, g0, g1, s0, s1):
    wid = lax.axis_index("s") * NC + lax.axis_index("c")
    base = wid * BPW
    rows = (r0, r1)
    gsem = (g0, g1)
    ssem = (s0, s1)
    pltpu.sync_copy(idx_hbm.at[pl.ds(base, BPW)], idx_v)

    def start_gather(i):
        return pltpu.async_copy(
            table_hbm.at[idx_v.at[pl.ds(i * CHUNK, CHUNK)]], rows[i % 2],
            gsem[i % 2])

    def start_store(i):
        return pltpu.async_copy(
            rows[i % 2], out_hbm.at[pl.ds(base + i * CHUNK, CHUNK)],
            ssem[i % 2])

    gathers = [None] * NCHUNK
    stores = [None] * NCHUNK
    gathers[0] = start_gather(0)
    for i in range(NCHUNK):
        if i + 1 < NCHUNK:
            if i >= 1:
                stores[i - 1].wait()
            gathers[i + 1] = start_gather(i + 1)
        gathers[i].wait()
        stores[i] = start_store(i)
    stores[NCHUNK - 2].wait()
    stores[NCHUNK - 1].wait()


def kernel(ids, table):
    flat = ids.reshape(B)
    out = _gather_rows(flat, table)
    return out.reshape(B0, H, D)
